# initial kernel scaffold (unmeasured)
import jax
import jax.numpy as jnp
from jax import lax
from jax.experimental import pallas as pl
from jax.experimental.pallas import tpu as pltpu

N_DEV = 4


def kernel(x, Win0, Wout0, Win1, Wout1, Win2, Wout2):
    b, d = x.shape
    rows_out = b // N_DEV

    def body(x_ref, win0_ref, wout0_ref, win1_ref, wout1_ref, win2_ref,
             wout2_ref, out_ref, comm_ar, p2_ref, comm_rs, send_sems,
             recv_sems):
        my_pos = lax.axis_index("i")

        def mlp_partial(xv, win_ref, wout_ref):
            hv = jnp.maximum(
                jnp.dot(xv, win_ref[...], preferred_element_type=jnp.float32),
                0.0,
            )
            return jnp.dot(hv, wout_ref[...],
                           preferred_element_type=jnp.float32)

        xv = x_ref[...]
        for r, (win_ref, wout_ref) in enumerate(
            [(win0_ref, wout0_ref), (win1_ref, wout1_ref)]
        ):
            comm_ar[r, 0] = mlp_partial(xv, win_ref, wout_ref)
            rdmas = []
            for k in range(1, N_DEV):
                dst_dev = lax.rem(my_pos + k, N_DEV)
                rdma = pltpu.make_async_remote_copy(
                    src_ref=comm_ar.at[r, 0],
                    dst_ref=comm_ar.at[r, k],
                    send_sem=send_sems.at[r, k],
                    recv_sem=recv_sems.at[r, k],
                    device_id=(dst_dev,),
                    device_id_type=pl.DeviceIdType.MESH,
                )
                rdma.start()
                rdmas.append(rdma)
            for rdma in rdmas:
                rdma.wait()
            xv = (comm_ar[r, 0] + comm_ar[r, 1]
                  + comm_ar[r, 2] + comm_ar[r, 3])

        p2_ref[...] = mlp_partial(xv, win2_ref, wout2_ref)
        rdmas = []
        for k in range(1, N_DEV):
            dst_dev = lax.rem(my_pos + k, N_DEV)
            rdma = pltpu.make_async_remote_copy(
                src_ref=p2_ref.at[pl.ds(dst_dev * rows_out, rows_out), :],
                dst_ref=comm_rs.at[k],
                send_sem=send_sems.at[2, k],
                recv_sem=recv_sems.at[2, k],
                device_id=(dst_dev,),
                device_id_type=pl.DeviceIdType.MESH,
            )
            rdma.start()
            rdmas.append(rdma)
        for rdma in rdmas:
            rdma.wait()
        own = p2_ref[pl.ds(my_pos * rows_out, rows_out), :]
        out_ref[...] = own + comm_rs[1] + comm_rs[2] + comm_rs[3]

    return pl.pallas_call(
        body,
        out_shape=jax.ShapeDtypeStruct((rows_out, d), jnp.float32),
        in_specs=[pl.BlockSpec(memory_space=pltpu.VMEM)] * 7,
        out_specs=pl.BlockSpec(memory_space=pltpu.VMEM),
        scratch_shapes=[
            pltpu.VMEM((2, N_DEV, b, d), jnp.float32),
            pltpu.VMEM((b, d), jnp.float32),
            pltpu.VMEM((N_DEV, rows_out, d), jnp.float32),
            pltpu.SemaphoreType.DMA((3, N_DEV)),
            pltpu.SemaphoreType.DMA((3, N_DEV)),
        ],
        compiler_params=pltpu.CompilerParams(collective_id=0),
    )(x, Win0, Wout0, Win1, Wout1, Win2, Wout2)


# baseline (device time: 33632 ns/iter reference)
import jax
import jax.numpy as jnp
from jax import lax
from jax.experimental import pallas as pl
from jax.experimental.pallas import tpu as pltpu

N_DEV = 4


def kernel(x, Win0, Wout0, Win1, Wout1, Win2, Wout2):
    b, d = x.shape
    rows_out = b // N_DEV

    def body(x_ref, win0_ref, wout0_ref, win1_ref, wout1_ref, win2_ref,
             wout2_ref, out_ref, comm_ar, p2_ref, comm_rs, send_sems,
             recv_sems):
        my_pos = lax.axis_index("i")

        def mlp_partial(xv, win_ref, wout_ref):
            hv = jnp.maximum(
                jnp.dot(xv, win_ref[...], preferred_element_type=jnp.float32),
                0.0,
            )
            return jnp.dot(hv, wout_ref[...],
                           preferred_element_type=jnp.float32)

        xv = x_ref[...]
        for r, (win_ref, wout_ref) in enumerate(
            [(win0_ref, wout0_ref), (win1_ref, wout1_ref)]
        ):
            comm_ar[r, 0] = mlp_partial(xv, win_ref, wout_ref)
            rdmas = []
            for k in range(1, N_DEV):
                dst_dev = lax.rem(my_pos + k, N_DEV)
                rdma = pltpu.make_async_remote_copy(
                    src_ref=comm_ar.at[r, 0],
                    dst_ref=comm_ar.at[r, k],
                    send_sem=send_sems.at[r, k],
                    recv_sem=recv_sems.at[r, k],
                    device_id=(dst_dev,),
                    device_id_type=pl.DeviceIdType.MESH,
                )
                rdma.start()
                rdmas.append(rdma)
            for rdma in rdmas:
                rdma.wait()
            xv = (comm_ar[r, 0] + comm_ar[r, 1]
                  + comm_ar[r, 2] + comm_ar[r, 3])

        p2_ref[...] = mlp_partial(xv, win2_ref, wout2_ref)
        rdmas = []
        for k in range(1, N_DEV):
            dst_dev = lax.rem(my_pos + k, N_DEV)
            rdma = pltpu.make_async_remote_copy(
                src_ref=p2_ref.at[pl.ds(dst_dev * rows_out, rows_out), :],
                dst_ref=comm_rs.at[k],
                send_sem=send_sems.at[2, k],
                recv_sem=recv_sems.at[2, k],
                device_id=(dst_dev,),
                device_id_type=pl.DeviceIdType.MESH,
            )
            rdma.start()
            rdmas.append(rdma)
        for rdma in rdmas:
            rdma.wait()
        own = p2_ref[pl.ds(my_pos * rows_out, rows_out), :]
        out_ref[...] = own + comm_rs[1] + comm_rs[2] + comm_rs[3]

    return pl.pallas_call(
        body,
        out_shape=jax.ShapeDtypeStruct((rows_out, d), jnp.float32),
        in_specs=[pl.BlockSpec(memory_space=pltpu.VMEM)] * 7,
        out_specs=pl.BlockSpec(memory_space=pltpu.VMEM),
        scratch_shapes=[
            pltpu.VMEM((2, N_DEV, b, d), jnp.float32),
            pltpu.VMEM((b, d), jnp.float32),
            pltpu.VMEM((N_DEV, rows_out, d), jnp.float32),
            pltpu.SemaphoreType.DMA((3, N_DEV)),
            pltpu.SemaphoreType.DMA((3, N_DEV)),
        ],
    )(x, Win0, Wout0, Win1, Wout1, Win2, Wout2)


# device time: 32295 ns/iter; 1.0414x vs baseline; 1.0414x over previous
import jax
import jax.numpy as jnp
from jax import lax
from jax.experimental import pallas as pl
from jax.experimental.pallas import tpu as pltpu

N_DEV = 4


def kernel(x, Win0, Wout0, Win1, Wout1, Win2, Wout2):
    b_rows, d = x.shape
    rb = b_rows // N_DEV

    def body(x_ref, win0_ref, wout0_ref, win1_ref, wout1_ref, win2_ref,
             wout2_ref, out_ref, stage, rs_buf, ag_buf, ag_src,
             rs_send, rs_recv, ag_send, ag_recv):
        my = lax.axis_index("i")
        pending = []

        def mlp(xv, win_ref, wout_ref):
            hv = jnp.maximum(
                jnp.dot(xv, win_ref[...], preferred_element_type=jnp.float32),
                0.0,
            )
            return jnp.dot(hv, wout_ref[...],
                           preferred_element_type=jnp.float32)

        def rs_send_block(bnd, k, val):
            stage[bnd, k - 1] = val
            dst = lax.rem(my + k, N_DEV)
            r = pltpu.make_async_remote_copy(
                src_ref=stage.at[bnd, k - 1],
                dst_ref=rs_buf.at[bnd, k],
                send_sem=rs_send.at[bnd, k],
                recv_sem=rs_recv.at[bnd, k],
                device_id=(dst,),
                device_id_type=pl.DeviceIdType.MESH,
            )
            r.start()
            pending.append(r)
            return r

        rs0 = []
        for k in range(1, N_DEV):
            dst = lax.rem(my + k, N_DEV)
            xb = x_ref[pl.ds(dst * rb, rb), :]
            rs0.append(rs_send_block(0, k, mlp(xb, win0_ref, wout0_ref)))
        p_own = mlp(x_ref[pl.ds(my * rb, rb), :], win0_ref, wout0_ref)

        for bnd, (win_ref, wout_ref) in enumerate(
            [(win1_ref, wout1_ref), (win2_ref, wout2_ref)]
        ):
            rs_prev = rs0 if bnd == 0 else rs_next
            for r in rs_prev:
                r.wait_recv()
            x_own = (p_own + rs_buf[bnd, 1] + rs_buf[bnd, 2]
                     + rs_buf[bnd, 3])
            ag_src[bnd] = x_own
            ags = []
            for k in range(1, N_DEV):
                dst = lax.rem(my + k, N_DEV)
                r = pltpu.make_async_remote_copy(
                    src_ref=ag_src.at[bnd],
                    dst_ref=ag_buf.at[bnd, k],
                    send_sem=ag_send.at[bnd, k],
                    recv_sem=ag_recv.at[bnd, k],
                    device_id=(dst,),
                    device_id_type=pl.DeviceIdType.MESH,
                )
                r.start()
                pending.append(r)
                ags.append(r)
            p_own = mlp(x_own, win_ref, wout_ref)
            rs_next = []
            for k in range(1, N_DEV):
                ags[k - 1].wait_recv()
                blk = mlp(ag_buf[bnd, k], win_ref, wout_ref)
                rs_next.append(rs_send_block(bnd + 1, N_DEV - k, blk))

        for r in rs_next:
            r.wait_recv()
        out_ref[...] = (p_own + rs_buf[2, 1] + rs_buf[2, 2]
                        + rs_buf[2, 3])

        for r in pending:
            r.wait_send()

    return pl.pallas_call(
        body,
        out_shape=jax.ShapeDtypeStruct((rb, d), jnp.float32),
        in_specs=[pl.BlockSpec(memory_space=pltpu.VMEM)] * 7,
        out_specs=pl.BlockSpec(memory_space=pltpu.VMEM),
        scratch_shapes=[
            pltpu.VMEM((3, N_DEV - 1, rb, d), jnp.float32),
            pltpu.VMEM((3, N_DEV, rb, d), jnp.float32),
            pltpu.VMEM((2, N_DEV, rb, d), jnp.float32),
            pltpu.VMEM((2, rb, d), jnp.float32),
            pltpu.SemaphoreType.DMA((3, N_DEV)),
            pltpu.SemaphoreType.DMA((3, N_DEV)),
            pltpu.SemaphoreType.DMA((2, N_DEV)),
            pltpu.SemaphoreType.DMA((2, N_DEV)),
        ],
    )(x, Win0, Wout0, Win1, Wout1, Win2, Wout2)


# device time: 29924 ns/iter; 1.1239x vs baseline; 1.0792x over previous
import jax
import jax.numpy as jnp
from jax import lax
from jax.experimental import pallas as pl
from jax.experimental.pallas import tpu as pltpu

N_DEV = 4


def kernel(x, Win0, Wout0, Win1, Wout1, Win2, Wout2):
    b_rows, d = x.shape
    rb = b_rows // N_DEV

    def body(x_ref, win0_ref, wout0_ref, win1_ref, wout1_ref, win2_ref,
             wout2_ref, out_ref, stage, rs_buf, ag_buf, ag_src,
             rs_send, rs_recv, ag_send, ag_recv):
        my = lax.axis_index("i")
        pending = []

        barrier_sem = pltpu.get_barrier_semaphore()
        for k in range(1, N_DEV):
            pl.semaphore_signal(
                barrier_sem, inc=1,
                device_id=(lax.rem(my + k, N_DEV),),
                device_id_type=pl.DeviceIdType.MESH,
            )
        pl.semaphore_wait(barrier_sem, N_DEV - 1)

        def mlp(xv, win_ref, wout_ref):
            hv = jnp.maximum(
                jnp.dot(xv, win_ref[...], preferred_element_type=jnp.float32),
                0.0,
            )
            return jnp.dot(hv, wout_ref[...],
                           preferred_element_type=jnp.float32)

        def rs_send_block(bnd, k, val):
            stage[bnd, k - 1] = val
            dst = lax.rem(my + k, N_DEV)
            r = pltpu.make_async_remote_copy(
                src_ref=stage.at[bnd, k - 1],
                dst_ref=rs_buf.at[bnd, k],
                send_sem=rs_send.at[bnd, k],
                recv_sem=rs_recv.at[bnd, k],
                device_id=(dst,),
                device_id_type=pl.DeviceIdType.MESH,
            )
            r.start()
            pending.append(r)
            return r

        rs0 = []
        for k in range(1, N_DEV):
            dst = lax.rem(my + k, N_DEV)
            xb = x_ref[pl.ds(dst * rb, rb), :]
            rs0.append(rs_send_block(0, k, mlp(xb, win0_ref, wout0_ref)))
        p_own = mlp(x_ref[pl.ds(my * rb, rb), :], win0_ref, wout0_ref)

        for bnd, (win_ref, wout_ref) in enumerate(
            [(win1_ref, wout1_ref), (win2_ref, wout2_ref)]
        ):
            rs_prev = rs0 if bnd == 0 else rs_next
            for r in rs_prev:
                r.wait_recv()
            x_own = (p_own + rs_buf[bnd, 1] + rs_buf[bnd, 2]
                     + rs_buf[bnd, 3])
            ag_src[bnd] = x_own
            ags = []
            for k in range(1, N_DEV):
                dst = lax.rem(my + k, N_DEV)
                r = pltpu.make_async_remote_copy(
                    src_ref=ag_src.at[bnd],
                    dst_ref=ag_buf.at[bnd, k],
                    send_sem=ag_send.at[bnd, k],
                    recv_sem=ag_recv.at[bnd, k],
                    device_id=(dst,),
                    device_id_type=pl.DeviceIdType.MESH,
                )
                r.start()
                pending.append(r)
                ags.append(r)
            p_own = mlp(x_own, win_ref, wout_ref)
            rs_next = []
            for k in range(1, N_DEV):
                ags[k - 1].wait_recv()
                blk = mlp(ag_buf[bnd, k], win_ref, wout_ref)
                rs_next.append(rs_send_block(bnd + 1, N_DEV - k, blk))

        for r in rs_next:
            r.wait_recv()
        out_ref[...] = (p_own + rs_buf[2, 1] + rs_buf[2, 2]
                        + rs_buf[2, 3])

        for r in pending:
            r.wait_send()

    return pl.pallas_call(
        body,
        out_shape=jax.ShapeDtypeStruct((rb, d), jnp.float32),
        in_specs=[pl.BlockSpec(memory_space=pltpu.VMEM)] * 7,
        out_specs=pl.BlockSpec(memory_space=pltpu.VMEM),
        scratch_shapes=[
            pltpu.VMEM((3, N_DEV - 1, rb, d), jnp.float32),
            pltpu.VMEM((3, N_DEV, rb, d), jnp.float32),
            pltpu.VMEM((2, N_DEV, rb, d), jnp.float32),
            pltpu.VMEM((2, rb, d), jnp.float32),
            pltpu.SemaphoreType.DMA((3, N_DEV)),
            pltpu.SemaphoreType.DMA((3, N_DEV)),
            pltpu.SemaphoreType.DMA((2, N_DEV)),
            pltpu.SemaphoreType.DMA((2, N_DEV)),
        ],
        compiler_params=pltpu.CompilerParams(collective_id=0),
    )(x, Win0, Wout0, Win1, Wout1, Win2, Wout2)


# device time: 26797 ns/iter; 1.2551x vs baseline; 1.1167x over previous
import jax
import jax.numpy as jnp
from jax import lax
from jax.experimental import pallas as pl
from jax.experimental.pallas import tpu as pltpu

N_DEV = 4


def kernel(x, Win0, Wout0, Win1, Wout1, Win2, Wout2):
    b_rows, d = x.shape
    rb = b_rows // N_DEV

    def body(x_ref, win0_ref, wout0_ref, win1_ref, wout1_ref, win2_ref,
             wout2_ref, out_ref, stage, rs_buf, ag_buf, ag_src,
             rs_send, rs_recv, ag_send, ag_recv):
        my = lax.axis_index("i")
        pending = []

        barrier_sem = pltpu.get_barrier_semaphore()
        for k in range(1, N_DEV):
            pl.semaphore_signal(
                barrier_sem, inc=1,
                device_id=(lax.rem(my + k, N_DEV),),
                device_id_type=pl.DeviceIdType.MESH,
            )
        pl.semaphore_wait(barrier_sem, N_DEV - 1)

        def mlp(xv, win_ref, wout_ref):
            hv = jnp.maximum(
                jnp.dot(xv, win_ref[...], preferred_element_type=jnp.float32),
                0.0,
            )
            return jnp.dot(hv, wout_ref[...],
                           preferred_element_type=jnp.float32)

        def rs_send_block(bnd, k, val):
            stage[bnd, k - 1] = val.astype(jnp.bfloat16)
            dst = lax.rem(my + k, N_DEV)
            r = pltpu.make_async_remote_copy(
                src_ref=stage.at[bnd, k - 1],
                dst_ref=rs_buf.at[bnd, k],
                send_sem=rs_send.at[bnd, k],
                recv_sem=rs_recv.at[bnd, k],
                device_id=(dst,),
                device_id_type=pl.DeviceIdType.MESH,
            )
            r.start()
            pending.append(r)
            return r

        rs0 = []
        for k in range(1, N_DEV):
            dst = lax.rem(my + k, N_DEV)
            xb = x_ref[pl.ds(dst * rb, rb), :]
            rs0.append(rs_send_block(0, k, mlp(xb, win0_ref, wout0_ref)))
        p_own = mlp(x_ref[pl.ds(my * rb, rb), :], win0_ref, wout0_ref)

        for bnd, (win_ref, wout_ref) in enumerate(
            [(win1_ref, wout1_ref), (win2_ref, wout2_ref)]
        ):
            rs_prev = rs0 if bnd == 0 else rs_next
            for r in rs_prev:
                r.wait_recv()
            x_own = p_own + (rs_buf[bnd, 1].astype(jnp.float32)
                             + rs_buf[bnd, 2].astype(jnp.float32)
                             + rs_buf[bnd, 3].astype(jnp.float32))
            ag_src[bnd] = x_own.astype(jnp.bfloat16)
            ags = []
            for k in range(1, N_DEV):
                dst = lax.rem(my + k, N_DEV)
                r = pltpu.make_async_remote_copy(
                    src_ref=ag_src.at[bnd],
                    dst_ref=ag_buf.at[bnd, k],
                    send_sem=ag_send.at[bnd, k],
                    recv_sem=ag_recv.at[bnd, k],
                    device_id=(dst,),
                    device_id_type=pl.DeviceIdType.MESH,
                )
                r.start()
                pending.append(r)
                ags.append(r)
            p_own = mlp(x_own, win_ref, wout_ref)
            rs_next = []
            for k in range(1, N_DEV):
                ags[k - 1].wait_recv()
                blk = mlp(ag_buf[bnd, k].astype(jnp.float32),
                          win_ref, wout_ref)
                rs_next.append(rs_send_block(bnd + 1, N_DEV - k, blk))

        for r in rs_next:
            r.wait_recv()
        out_ref[...] = p_own + (rs_buf[2, 1].astype(jnp.float32)
                                + rs_buf[2, 2].astype(jnp.float32)
                                + rs_buf[2, 3].astype(jnp.float32))

        for r in pending:
            r.wait_send()

    return pl.pallas_call(
        body,
        out_shape=jax.ShapeDtypeStruct((rb, d), jnp.float32),
        in_specs=[pl.BlockSpec(memory_space=pltpu.VMEM)] * 7,
        out_specs=pl.BlockSpec(memory_space=pltpu.VMEM),
        scratch_shapes=[
            pltpu.VMEM((3, N_DEV - 1, rb, d), jnp.bfloat16),
            pltpu.VMEM((3, N_DEV, rb, d), jnp.bfloat16),
            pltpu.VMEM((2, N_DEV, rb, d), jnp.bfloat16),
            pltpu.VMEM((2, rb, d), jnp.bfloat16),
            pltpu.SemaphoreType.DMA((3, N_DEV)),
            pltpu.SemaphoreType.DMA((3, N_DEV)),
            pltpu.SemaphoreType.DMA((2, N_DEV)),
            pltpu.SemaphoreType.DMA((2, N_DEV)),
        ],
        compiler_params=pltpu.CompilerParams(collective_id=0),
    )(x, Win0, Wout0, Win1, Wout1, Win2, Wout2)


# device time: 25146 ns/iter; 1.3375x vs baseline; 1.0657x over previous
import jax
import jax.numpy as jnp
from jax import lax
from jax.experimental import pallas as pl
from jax.experimental.pallas import tpu as pltpu

N_DEV = 4


def kernel(x, Win0, Wout0, Win1, Wout1, Win2, Wout2):
    b_rows, d = x.shape
    rb = b_rows // N_DEV

    def body(x_ref, win0_ref, wout0_ref, win1_ref, wout1_ref, win2_ref,
             wout2_ref, out_ref, stage, rs_buf, ag_buf, ag_src,
             rs_send, rs_recv, ag_send, ag_recv):
        my = lax.axis_index("i")
        pending = []

        barrier_sem = pltpu.get_barrier_semaphore()
        for k in range(1, N_DEV):
            pl.semaphore_signal(
                barrier_sem, inc=1,
                device_id=(lax.rem(my + k, N_DEV),),
                device_id_type=pl.DeviceIdType.MESH,
            )
        pl.semaphore_wait(barrier_sem, N_DEV - 1)

        def mlp(xv, win_ref, wout_ref):
            hv = jnp.maximum(
                jnp.dot(xv, win_ref[...], preferred_element_type=jnp.float32),
                0.0,
            )
            return jnp.dot(hv, wout_ref[...],
                           preferred_element_type=jnp.float32)

        def rs_send_block(bnd, k, val):
            stage[bnd, k - 1] = val.astype(jnp.bfloat16)
            dst = lax.rem(my + k, N_DEV)
            r = pltpu.make_async_remote_copy(
                src_ref=stage.at[bnd, k - 1],
                dst_ref=rs_buf.at[bnd, k],
                send_sem=rs_send.at[bnd, k],
                recv_sem=rs_recv.at[bnd, k],
                device_id=(dst,),
                device_id_type=pl.DeviceIdType.MESH,
            )
            r.start()
            pending.append(r)
            return r

        rs0 = []
        for k in range(1, N_DEV):
            dst = lax.rem(my + k, N_DEV)
            xb = x_ref[pl.ds(dst * rb, rb), :]
            rs0.append(rs_send_block(0, k, mlp(xb, win0_ref, wout0_ref)))
        p_local = mlp(x_ref[pl.ds(my * rb, rb), :], win0_ref, wout0_ref)

        for bnd, (win_ref, wout_ref) in enumerate(
            [(win1_ref, wout1_ref), (win2_ref, wout2_ref)]
        ):
            rs_prev = rs0 if bnd == 0 else rs_next
            for r in rs_prev:
                r.wait_recv()
            x_own = p_local + (rs_buf[bnd, 1].astype(jnp.float32)
                               + rs_buf[bnd, 2].astype(jnp.float32)
                               + rs_buf[bnd, 3].astype(jnp.float32))
            ag_src[bnd] = x_own.astype(jnp.bfloat16)
            ags = []
            for k in range(1, N_DEV):
                dst = lax.rem(my + k, N_DEV)
                r = pltpu.make_async_remote_copy(
                    src_ref=ag_src.at[bnd],
                    dst_ref=ag_buf.at[bnd, k],
                    send_sem=ag_send.at[bnd, k],
                    recv_sem=ag_recv.at[bnd, k],
                    device_id=(dst,),
                    device_id_type=pl.DeviceIdType.MESH,
                )
                r.start()
                pending.append(r)
                ags.append(r)
            rs_next = []
            rs_next.append(
                rs_send_block(bnd + 1, 2, mlp(x_own, win_ref, wout_ref)))
            for k in (1, 3):
                ags[k - 1].wait_recv()
                blk = mlp(ag_buf[bnd, k].astype(jnp.float32),
                          win_ref, wout_ref)
                rs_next.append(rs_send_block(bnd + 1, k, blk))
            ags[1].wait_recv()
            p_local = mlp(ag_buf[bnd, 2].astype(jnp.float32),
                          win_ref, wout_ref)

        for r in rs_next:
            r.wait_recv()
        out_ref[...] = p_local + (rs_buf[2, 1].astype(jnp.float32)
                                  + rs_buf[2, 2].astype(jnp.float32)
                                  + rs_buf[2, 3].astype(jnp.float32))

        for r in pending:
            r.wait_send()

    return pl.pallas_call(
        body,
        out_shape=jax.ShapeDtypeStruct((rb, d), jnp.float32),
        in_specs=[pl.BlockSpec(memory_space=pltpu.VMEM)] * 7,
        out_specs=pl.BlockSpec(memory_space=pltpu.VMEM),
        scratch_shapes=[
            pltpu.VMEM((3, N_DEV - 1, rb, d), jnp.bfloat16),
            pltpu.VMEM((3, N_DEV, rb, d), jnp.bfloat16),
            pltpu.VMEM((2, N_DEV, rb, d), jnp.bfloat16),
            pltpu.VMEM((2, rb, d), jnp.bfloat16),
            pltpu.SemaphoreType.DMA((3, N_DEV)),
            pltpu.SemaphoreType.DMA((3, N_DEV)),
            pltpu.SemaphoreType.DMA((2, N_DEV)),
            pltpu.SemaphoreType.DMA((2, N_DEV)),
        ],
        compiler_params=pltpu.CompilerParams(collective_id=0),
    )(x, Win0, Wout0, Win1, Wout1, Win2, Wout2)
